# baseline (speedup 1.0000x reference)
import functools

import jax
import jax.numpy as jnp
from jax import lax
from jax.experimental import pallas as pl
from jax.experimental.pallas import tpu as pltpu
from jax.experimental.pallas import tpu_sc as plsc

V, H, E = 32, 1024, 1536
N = 64
ROW = H * E
N_SC = 14
N_TC = N - N_SC

R = 32
D = ROW // R
NW = 32
B_TOTAL = N_SC * R
B_W = B_TOTAL // NW
NP = B_W // 2

_mesh = plsc.VectorSubcoreMesh(core_axis_name="c", subcore_axis_name="s")


@functools.partial(
    pl.kernel,
    mesh=_mesh,
    out_type=jax.ShapeDtypeStruct((B_TOTAL, D), jnp.float32),
    scratch_types=[
        pltpu.VMEM((1, B_W, 8), jnp.int32),
        pltpu.VMEM((1, D), jnp.float32),
        pltpu.VMEM((1, D), jnp.float32),
        pltpu.SemaphoreType.DMA,
        pltpu.SemaphoreType.DMA,
        pltpu.SemaphoreType.DMA,
        pltpu.SemaphoreType.DMA,
    ],
)
def _sc_gather(table_hbm, idx_hbm, out_hbm, idx_v, buf0, buf1, g0, g1, w0, w1):
    wid = lax.axis_index("s") * 2 + lax.axis_index("c")
    base = wid * B_W
    pltpu.sync_copy(idx_hbm.at[pl.ds(wid, 1)], idx_v)

    def gather(j, buf, sem):
        pltpu.async_copy(table_hbm.at[idx_v.at[0, j, pl.ds(0, 1)]], buf, sem)

    def write(j, buf, sem):
        pltpu.async_copy(buf, out_hbm.at[pl.ds(base + j, 1)], sem)

    def wait_gather(buf, sem):
        pltpu.make_async_copy(table_hbm.at[pl.ds(0, 1)], buf, sem).wait()

    def wait_write(buf, sem):
        pltpu.make_async_copy(buf, out_hbm.at[pl.ds(base, 1)], sem).wait()

    gather(0, buf0, g0)

    def body(p, carry):
        j0 = 2 * p
        wait_gather(buf0, g0)
        write(j0, buf0, w0)

        @pl.when(p > 0)
        def _():
            wait_write(buf1, w1)

        gather(j0 + 1, buf1, g1)

        wait_gather(buf1, g1)
        write(j0 + 1, buf1, w1)
        wait_write(buf0, w0)

        @pl.when(p < NP - 1)
        def _():
            gather(j0 + 2, buf0, g0)

        return carry

    lax.fori_loop(0, NP, body, 0)
    wait_write(buf1, w1)


def _copy_body(ids_smem, in_ref, out_ref):
    out_ref[...] = in_ref[...]


def _tc_gather(ids, table):
    return pl.pallas_call(
        _copy_body,
        grid_spec=pltpu.PrefetchScalarGridSpec(
            num_scalar_prefetch=1,
            grid=(N_TC,),
            in_specs=[
                pl.BlockSpec((1, H, E), lambda i, ids: (ids[i], 0, 0)),
            ],
            out_specs=pl.BlockSpec((1, H, E), lambda i, ids: (i, 0, 0)),
        ),
        out_shape=jax.ShapeDtypeStruct((N_TC, H, E), jnp.float32),
    )(ids, table)


def kernel(cat_ids, W):
    ids = cat_ids.astype(jnp.int32)
    table = W.reshape(V * R, D)
    idx = (ids[:N_SC, None] * R
           + jnp.arange(R, dtype=jnp.int32)[None, :]).reshape(-1)
    idx8 = jnp.broadcast_to(idx[:, None], (B_TOTAL, 8)).reshape(NW, B_W, 8)
    out_sc = _sc_gather(table, idx8).reshape(N_SC, H, E)
    out_tc = _tc_gather(ids[N_SC:], W)
    return jnp.concatenate([out_sc, out_tc], axis=0)


# speedup vs baseline: 3.9658x; 3.9658x over previous
import jax
import jax.numpy as jnp
from jax.experimental import pallas as pl
from jax.experimental.pallas import tpu as pltpu

V, H, E = 32, 1024, 1536
N = 64


def _copy_body(sids_smem, order_smem, in_ref, out_ref):
    out_ref[...] = in_ref[...]


def _tc_gather(sids, order, table):
    return pl.pallas_call(
        _copy_body,
        grid_spec=pltpu.PrefetchScalarGridSpec(
            num_scalar_prefetch=2,
            grid=(N,),
            in_specs=[
                pl.BlockSpec((1, H, E), lambda i, sids, order: (sids[i], 0, 0)),
            ],
            out_specs=pl.BlockSpec((1, H, E), lambda i, sids, order: (order[i], 0, 0)),
        ),
        out_shape=jax.ShapeDtypeStruct((N, H, E), jnp.float32),
    )(sids, order, table)


def kernel(cat_ids, W):
    ids = cat_ids.astype(jnp.int32)
    order = jnp.argsort(ids).astype(jnp.int32)
    sids = ids[order]
    return _tc_gather(sids, order, W)
